# baseline (speedup 1.0000x reference)
import functools

import jax
import jax.numpy as jnp
from jax import lax
from jax.experimental import pallas as pl
from jax.experimental.pallas import tpu as pltpu
from jax.experimental.pallas import tpu_sc as plsc

N = 50000
E = 800000
IN = 128
WS = 64
OUT = 64

NT = 50432
EPAD = 819200
PAD = EPAD - E
ROWS = EPAD // 128

NC = 2
NS = 16
HALF = N // 2
OT = 25088
ZSL = NT // NS

BLK = 1000


def _k0_body(h_ref, wh_ref, wg0_ref, wg1_ref, pv_ref, f0_ref, f1_ref, v_ref):
    wh = h_ref[...] @ wh_ref[...]
    f0 = wh @ wg0_ref[...]
    f1 = wh @ wg1_ref[...]
    f0_ref[...] = f0
    f1_ref[...] = f1
    p = pv_ref[...]
    el0 = jnp.sum(f0 * p[0][None, :], axis=1)
    el1 = jnp.sum(f1 * p[1][None, :], axis=1)
    er0 = jnp.sum(f0 * p[2][None, :], axis=1)
    er1 = jnp.sum(f1 * p[3][None, :], axis=1)
    es0 = el0 + er0
    es1 = el1 + er1
    es0 = jnp.exp(jnp.where(es0 > 0, es0, 0.2 * es0))
    es1 = jnp.exp(jnp.where(es1 > 0, es1, 0.2 * es1))
    z = jnp.zeros_like(el0)
    v_ref[...] = jnp.stack([el0, er0, el1, er1, es0, es1, z, z], axis=1)


def _k0(h, Wh, Wg0, Wg1, pvec):
    return pl.pallas_call(
        _k0_body,
        out_shape=(
            jax.ShapeDtypeStruct((N, OUT), jnp.float32),
            jax.ShapeDtypeStruct((N, OUT), jnp.float32),
            jax.ShapeDtypeStruct((N, 8), jnp.float32),
        ),
        grid=(N // BLK,),
        in_specs=[
            pl.BlockSpec((BLK, IN), lambda i: (i, 0)),
            pl.BlockSpec((IN, WS), lambda i: (0, 0)),
            pl.BlockSpec((WS, OUT), lambda i: (0, 0)),
            pl.BlockSpec((WS, OUT), lambda i: (0, 0)),
            pl.BlockSpec((8, OUT), lambda i: (0, 0)),
        ],
        out_specs=(
            pl.BlockSpec((BLK, OUT), lambda i: (i, 0)),
            pl.BlockSpec((BLK, OUT), lambda i: (i, 0)),
            pl.BlockSpec((BLK, 8), lambda i: (i, 0)),
        ),
    )(h, Wh, Wg0, Wg1, pvec)


def _k1_body(el_hbm, er_hbm, src_hbm, dst_hbm, eexp_hbm, ssum_hbm,
             el_v, er_v, srcb, dstb, eeb, zb, ssum_sh):
    cid = lax.axis_index("c")
    sid = lax.axis_index("s")

    def zloop(i, c):
        zb[pl.ds(i * 16, 16)] = jnp.zeros((16,), jnp.float32)
        return c

    lax.fori_loop(0, ZSL // 16, zloop, 0)
    pltpu.sync_copy(zb, ssum_sh.at[pl.ds(sid * ZSL, ZSL)])
    pltpu.sync_copy(el_hbm, el_v)
    pltpu.sync_copy(er_hbm, er_v)
    plsc.subcore_barrier()

    w = sid * NC + cid
    row0 = w * (ROWS // (NC * NS))

    def win(i, c):
        r = row0 + i * 4
        pltpu.sync_copy(src_hbm.at[pl.ds(r, 4)], srcb)
        pltpu.sync_copy(dst_hbm.at[pl.ds(r, 4)], dstb)
        for j in range(4):
            for k in range(8):
                s = srcb[j, pl.ds(k * 16, 16)]
                d = dstb[j, pl.ds(k * 16, 16)]
                e = plsc.load_gather(el_v, [s]) + plsc.load_gather(er_v, [d])
                e = jnp.where(e > 0, e, 0.2 * e)
                eeb[j, pl.ds(k * 16, 16)] = jnp.exp(e)
        pltpu.sync_copy(eeb, eexp_hbm.at[pl.ds(r, 4)])
        for j in range(4):
            pltpu.sync_copy(eeb.at[j], ssum_sh.at[dstb.at[j]], add=True)
        return c

    lax.fori_loop(0, (ROWS // (NC * NS)) // 4, win, 0)
    plsc.subcore_barrier()
    pltpu.sync_copy(ssum_sh.at[pl.ds(sid * ZSL, ZSL)], zb)
    pltpu.sync_copy(zb, ssum_hbm.at[pl.ds(cid * NT + sid * ZSL, ZSL)])


_k1 = functools.partial(
    pl.kernel,
    _k1_body,
    out_type=(
        jax.ShapeDtypeStruct((ROWS, 128), jnp.float32),
        jax.ShapeDtypeStruct((NC * NT,), jnp.float32),
    ),
    mesh=plsc.VectorSubcoreMesh(core_axis_name="c", subcore_axis_name="s"),
    compiler_params=pltpu.CompilerParams(needs_layout_passes=False),
    scratch_types=[
        pltpu.VMEM((NT,), jnp.float32),
        pltpu.VMEM((NT,), jnp.float32),
        pltpu.VMEM((4, 128), jnp.int32),
        pltpu.VMEM((4, 128), jnp.int32),
        pltpu.VMEM((4, 128), jnp.float32),
        pltpu.VMEM((ZSL,), jnp.float32),
        pltpu.VMEM_SHARED((NT,), jnp.float32),
    ],
)()


def _k2_body(feat_hbm, src_hbm, dst_hbm, ee_hbm, zraw_hbm,
             srcb, dstb, eeb, ldb, rows_v, out_sh,
             lsem, gsem, ssem):
    cid = lax.axis_index("c")
    sid = lax.axis_index("s")
    lo = cid * HALF

    def zloop(i, c):
        for cc in range(4):
            rows_v[i, pl.ds(cc * 16, 16)] = jnp.zeros((16,), jnp.float32)
        return c

    lax.fori_loop(0, 256, zloop, 0)
    for c in range(7):
        pltpu.sync_copy(rows_v.at[pl.ds(0, 224)],
                        out_sh.at[pl.ds(sid * (OT // NS) + c * 224, 224)])
    plsc.subcore_barrier()

    row0 = sid * (ROWS // NS)
    nwin = ROWS // NS

    def load_idx(i):
        b = i % 2
        pltpu.async_copy(src_hbm.at[pl.ds(row0 + i, 1)],
                         srcb.at[pl.ds(b, 1)], lsem)
        pltpu.async_copy(dst_hbm.at[pl.ds(row0 + i, 1)],
                         dstb.at[pl.ds(b, 1)], lsem)
        pltpu.async_copy(ee_hbm.at[pl.ds(row0 + i, 1)],
                         eeb.at[pl.ds(b, 1)], lsem)

    def wait_idx(i):
        b = i % 2
        pltpu.make_async_copy(src_hbm.at[pl.ds(row0, 1)],
                              srcb.at[pl.ds(b, 1)], lsem).wait()
        pltpu.make_async_copy(dst_hbm.at[pl.ds(row0, 1)],
                              dstb.at[pl.ds(b, 1)], lsem).wait()
        pltpu.make_async_copy(ee_hbm.at[pl.ds(row0, 1)],
                              eeb.at[pl.ds(b, 1)], lsem).wait()

    def fire_gather(i):
        b = i % 2
        for k in range(8):
            d = dstb[b, pl.ds(k * 16, 16)]
            s = srcb[b, pl.ds(k * 16, 16)]
            inb = (d >= lo) & (d < lo + HALF)
            ldb[i % 4, pl.ds(k * 16, 16)] = jnp.where(
                inb, d - lo, HALF + (s & 63))
        pltpu.async_copy(feat_hbm.at[srcb.at[b]],
                         rows_v.at[pl.ds((i % 3) * 128, 128)], gsem)

    def wait_gather(i):
        b = i % 2
        pltpu.make_async_copy(feat_hbm.at[srcb.at[b]],
                              rows_v.at[pl.ds((i % 3) * 128, 128)],
                              gsem).wait()

    def wait_scatter(i):
        pltpu.make_async_copy(rows_v.at[pl.ds((i % 3) * 128, 128)],
                              out_sh.at[ldb.at[i % 4]], ssem).wait()

    load_idx(0)
    wait_idx(0)
    fire_gather(0)

    def win(i, c):
        @pl.when(i + 1 < nwin)
        def _():
            load_idx(i + 1)
        wait_gather(i)

        @pl.when(i >= 2)
        def _():
            wait_scatter(i - 2)

        @pl.when(i + 1 < nwin)
        def _():
            wait_idx(i + 1)
            fire_gather(i + 1)
        b = i % 2
        r0 = (i % 3) * 128

        def scale(g, c2):
            for u in range(4):
                e = r0 + g * 4 + u
                a = plsc.load_gather(
                    eeb, [jnp.full((16,), b, jnp.int32),
                          jnp.full((16,), (g * 4 + u), jnp.int32)])
                for cc in range(4):
                    rows_v[e, pl.ds(cc * 16, 16)] = (
                        rows_v[e, pl.ds(cc * 16, 16)] * a)
            return c2

        lax.fori_loop(0, 32, scale, 0)
        pltpu.async_copy(rows_v.at[pl.ds(r0, 128)],
                         out_sh.at[ldb.at[i % 4]], ssem, add=True)
        return c

    lax.fori_loop(0, nwin, win, 0)
    wait_scatter(nwin - 2)
    wait_scatter(nwin - 1)
    plsc.subcore_barrier()
    for c in range(7):
        o = sid * (OT // NS) + c * 224
        pltpu.sync_copy(out_sh.at[pl.ds(o, 224)], rows_v.at[pl.ds(0, 224)])
        pltpu.sync_copy(rows_v.at[pl.ds(0, 224)],
                        zraw_hbm.at[cid, pl.ds(o, 224)])


_k2 = functools.partial(
    pl.kernel,
    _k2_body,
    out_type=jax.ShapeDtypeStruct((NC, OT, OUT), jnp.float32),
    mesh=plsc.VectorSubcoreMesh(core_axis_name="c", subcore_axis_name="s"),
    compiler_params=pltpu.CompilerParams(
        needs_layout_passes=False, use_tc_tiling_on_sc=False),
    scratch_types=[
        pltpu.VMEM((2, 128), jnp.int32),
        pltpu.VMEM((2, 128), jnp.int32),
        pltpu.VMEM((2, 128), jnp.float32),
        pltpu.VMEM((4, 128), jnp.int32),
        pltpu.VMEM((384, OUT), jnp.float32),
        pltpu.VMEM_SHARED((OT, OUT), jnp.float32),
        pltpu.SemaphoreType.DMA,
        pltpu.SemaphoreType.DMA,
        pltpu.SemaphoreType.DMA,
    ],
)()


def _k3_body(zr0_ref, zr1_ref, f0_ref, f1_ref, aux_ref, bg_ref,
             w1_ref, b1_ref, w2_ref, o_ref):
    aux = aux_ref[...]
    bgp = bg_ref[...]
    w2 = w2_ref[...]

    def path(zr_ref, f_ref, den, ees, bias):
        x = (zr_ref[0] + ees[:, None] * f_ref[...]) / den[:, None]
        x = x + bias[None, :]
        return jnp.where(x > 0, x, jnp.exp(jnp.minimum(x, 0.0)) - 1.0)

    z0 = path(zr0_ref, f0_ref, aux[:, 0], aux[:, 2], bgp[0])
    z1 = path(zr1_ref, f1_ref, aux[:, 1], aux[:, 3], bgp[1])
    t0 = jnp.tanh(z0 @ w1_ref[...] + b1_ref[...])
    t1 = jnp.tanh(z1 @ w1_ref[...] + b1_ref[...])
    w0 = jnp.sum(t0 * w2[0][None, :], axis=1)
    w1s = jnp.sum(t1 * w2[0][None, :], axis=1)
    m = jnp.maximum(w0, w1s)
    e0 = jnp.exp(w0 - m)
    e1 = jnp.exp(w1s - m)
    b0 = (e0 / (e0 + e1))[:, None]
    b1v = (e1 / (e0 + e1))[:, None]
    o_ref[...] = b0 * z0 + b1v * z1


def _k3(zr0, zr1, f0, f1, aux, bgp, W1, b1r, w2r):
    return pl.pallas_call(
        _k3_body,
        out_shape=jax.ShapeDtypeStruct((N, OUT), jnp.float32),
        grid=(N // BLK,),
        in_specs=[
            pl.BlockSpec((1, BLK, OUT), lambda i: (i // 25, i % 25, 0)),
            pl.BlockSpec((1, BLK, OUT), lambda i: (i // 25, i % 25, 0)),
            pl.BlockSpec((BLK, OUT), lambda i: (i, 0)),
            pl.BlockSpec((BLK, OUT), lambda i: (i, 0)),
            pl.BlockSpec((BLK, 8), lambda i: (i, 0)),
            pl.BlockSpec((8, OUT), lambda i: (0, 0)),
            pl.BlockSpec((OUT, 128), lambda i: (0, 0)),
            pl.BlockSpec((1, 128), lambda i: (0, 0)),
            pl.BlockSpec((1, 128), lambda i: (0, 0)),
        ],
        out_specs=pl.BlockSpec((BLK, OUT), lambda i: (i, 0)),
    )(zr0, zr1, f0, f1, aux, bgp, W1, b1r, w2r)


def kernel(h, edge_index0, edge_index1, Wh, Wg, al, ar, bg, W1, b1, W2):
    pad_i = jnp.arange(PAD, dtype=jnp.int32)
    pad_src = pad_i % 2048
    pad_dst = N + (pad_i % 256)

    def prep(ei):
        srcp = jnp.concatenate([ei[0], pad_src]).reshape(ROWS, 128)
        dstp = jnp.concatenate([ei[1], pad_dst]).reshape(ROWS, 128)
        return srcp, dstp

    src0, dst0 = prep(edge_index0)
    src1, dst1 = prep(edge_index1)

    pvec = jnp.concatenate(
        [al[:, 0, :], ar[:, 0, :], jnp.zeros((4, OUT), jnp.float32)], axis=0)

    feat0, feat1, vecs = _k0(h, Wh, Wg[0], Wg[1], pvec)

    elp0 = jnp.pad(vecs[:, 0], (0, NT - N))
    erp0 = jnp.pad(vecs[:, 1], (0, NT - N))
    elp1 = jnp.pad(vecs[:, 2], (0, NT - N))
    erp1 = jnp.pad(vecs[:, 3], (0, NT - N))

    ee0, sp0 = _k1(elp0, erp0, src0, dst0)
    ee1, sp1 = _k1(elp1, erp1, src1, dst1)

    zraw0 = _k2(feat0, src0, dst0, ee0)
    zraw1 = _k2(feat1, src1, dst1, ee1)

    den0 = sp0[:N] + sp0[NT:NT + N] + vecs[:, 4] + 1e-9
    den1 = sp1[:N] + sp1[NT:NT + N] + vecs[:, 5] + 1e-9
    aux = jnp.stack([den0, den1, vecs[:, 4], vecs[:, 5]], axis=1)
    aux = jnp.concatenate([aux, jnp.zeros((N, 4), jnp.float32)], axis=1)

    bgp = jnp.concatenate([bg, jnp.zeros((6, OUT), jnp.float32)], axis=0)
    return _k3(zraw0, zraw1, feat0, feat1, aux, bgp, W1,
               b1.reshape(1, 128), W2.reshape(1, 128))


# speedup vs baseline: 1.5294x; 1.5294x over previous
import functools

import jax
import jax.numpy as jnp
from jax import lax
from jax.experimental import pallas as pl
from jax.experimental.pallas import tpu as pltpu
from jax.experimental.pallas import tpu_sc as plsc

N = 50000
E = 800000
IN = 128
WS = 64
OUT = 64

NT = 50432
EPAD = 819200
PAD = EPAD - E
ROWS = EPAD // 128

NC = 2
NS = 16
HALF = N // 2
OT = 25088
ZSL = NT // NS

BLK = 1000


def _k0_body(h_ref, wh_ref, wg0_ref, wg1_ref, pv_ref, f0_ref, f1_ref, v_ref):
    wh = h_ref[...] @ wh_ref[...]
    f0 = wh @ wg0_ref[...]
    f1 = wh @ wg1_ref[...]
    f0_ref[...] = f0
    f1_ref[...] = f1
    p = pv_ref[...]
    el0 = jnp.sum(f0 * p[0][None, :], axis=1)
    el1 = jnp.sum(f1 * p[1][None, :], axis=1)
    er0 = jnp.sum(f0 * p[2][None, :], axis=1)
    er1 = jnp.sum(f1 * p[3][None, :], axis=1)
    es0 = el0 + er0
    es1 = el1 + er1
    es0 = jnp.exp(jnp.where(es0 > 0, es0, 0.2 * es0))
    es1 = jnp.exp(jnp.where(es1 > 0, es1, 0.2 * es1))
    z = jnp.zeros_like(el0)
    v_ref[...] = jnp.stack([el0, er0, el1, er1, es0, es1, z, z], axis=1)


def _k0(h, Wh, Wg0, Wg1, pvec):
    return pl.pallas_call(
        _k0_body,
        out_shape=(
            jax.ShapeDtypeStruct((N, OUT), jnp.float32),
            jax.ShapeDtypeStruct((N, OUT), jnp.float32),
            jax.ShapeDtypeStruct((N, 8), jnp.float32),
        ),
        grid=(N // BLK,),
        in_specs=[
            pl.BlockSpec((BLK, IN), lambda i: (i, 0)),
            pl.BlockSpec((IN, WS), lambda i: (0, 0)),
            pl.BlockSpec((WS, OUT), lambda i: (0, 0)),
            pl.BlockSpec((WS, OUT), lambda i: (0, 0)),
            pl.BlockSpec((8, OUT), lambda i: (0, 0)),
        ],
        out_specs=(
            pl.BlockSpec((BLK, OUT), lambda i: (i, 0)),
            pl.BlockSpec((BLK, OUT), lambda i: (i, 0)),
            pl.BlockSpec((BLK, 8), lambda i: (i, 0)),
        ),
    )(h, Wh, Wg0, Wg1, pvec)


def _k1_body(el_hbm, er_hbm, src_hbm, dst_hbm, eexp_hbm, ssum_hbm,
             el_v, er_v, srcb, dstb, eeb, zb, ssum_sh):
    cid = lax.axis_index("c")
    sid = lax.axis_index("s")

    def zloop(i, c):
        zb[pl.ds(i * 16, 16)] = jnp.zeros((16,), jnp.float32)
        return c

    lax.fori_loop(0, ZSL // 16, zloop, 0)
    pltpu.sync_copy(zb, ssum_sh.at[pl.ds(sid * ZSL, ZSL)])
    pltpu.sync_copy(el_hbm, el_v)
    pltpu.sync_copy(er_hbm, er_v)
    plsc.subcore_barrier()

    w = sid * NC + cid
    row0 = w * (ROWS // (NC * NS))

    def win(i, c):
        r = row0 + i * 4
        pltpu.sync_copy(src_hbm.at[pl.ds(r, 4)], srcb)
        pltpu.sync_copy(dst_hbm.at[pl.ds(r, 4)], dstb)
        for j in range(4):
            for k in range(8):
                s = srcb[j, pl.ds(k * 16, 16)]
                d = dstb[j, pl.ds(k * 16, 16)]
                e = plsc.load_gather(el_v, [s]) + plsc.load_gather(er_v, [d])
                e = jnp.where(e > 0, e, 0.2 * e)
                eeb[j, pl.ds(k * 16, 16)] = jnp.exp(e)
        pltpu.sync_copy(eeb, eexp_hbm.at[pl.ds(r, 4)])
        for j in range(4):
            pltpu.sync_copy(eeb.at[j], ssum_sh.at[dstb.at[j]], add=True)
        return c

    lax.fori_loop(0, (ROWS // (NC * NS)) // 4, win, 0)
    plsc.subcore_barrier()
    pltpu.sync_copy(ssum_sh.at[pl.ds(sid * ZSL, ZSL)], zb)
    pltpu.sync_copy(zb, ssum_hbm.at[pl.ds(cid * NT + sid * ZSL, ZSL)])


_k1 = functools.partial(
    pl.kernel,
    _k1_body,
    out_type=(
        jax.ShapeDtypeStruct((ROWS, 128), jnp.float32),
        jax.ShapeDtypeStruct((NC * NT,), jnp.float32),
    ),
    mesh=plsc.VectorSubcoreMesh(core_axis_name="c", subcore_axis_name="s"),
    compiler_params=pltpu.CompilerParams(needs_layout_passes=False),
    scratch_types=[
        pltpu.VMEM((NT,), jnp.float32),
        pltpu.VMEM((NT,), jnp.float32),
        pltpu.VMEM((4, 128), jnp.int32),
        pltpu.VMEM((4, 128), jnp.int32),
        pltpu.VMEM((4, 128), jnp.float32),
        pltpu.VMEM((ZSL,), jnp.float32),
        pltpu.VMEM_SHARED((NT,), jnp.float32),
    ],
)()


def _k2_body(feat_hbm, src_hbm, dst_hbm, ee_hbm, zraw_hbm,
             srcb, dstb, eeb, ldb, rows_v, out_sh,
             lsem, gsem):
    cid = lax.axis_index("c")
    sid = lax.axis_index("s")
    lo = cid * HALF

    def zloop(i, c):
        for cc in range(4):
            rows_v[i, pl.ds(cc * 16, 16)] = jnp.zeros((16,), jnp.float32)
        return c

    lax.fori_loop(0, 256, zloop, 0)
    for c in range(7):
        pltpu.sync_copy(rows_v.at[pl.ds(0, 224)],
                        out_sh.at[pl.ds(sid * (OT // NS) + c * 224, 224)])
    plsc.subcore_barrier()

    row0 = sid * (ROWS // NS)
    nwin = ROWS // NS

    def load_idx(i):
        b = i % 2
        pltpu.async_copy(src_hbm.at[pl.ds(row0 + i, 1)],
                         srcb.at[pl.ds(b, 1)], lsem)
        pltpu.async_copy(dst_hbm.at[pl.ds(row0 + i, 1)],
                         dstb.at[pl.ds(b, 1)], lsem)
        pltpu.async_copy(ee_hbm.at[pl.ds(row0 + i, 1)],
                         eeb.at[pl.ds(b, 1)], lsem)

    def wait_idx(i):
        b = i % 2
        pltpu.make_async_copy(src_hbm.at[pl.ds(row0, 1)],
                              srcb.at[pl.ds(b, 1)], lsem).wait()
        pltpu.make_async_copy(dst_hbm.at[pl.ds(row0, 1)],
                              dstb.at[pl.ds(b, 1)], lsem).wait()
        pltpu.make_async_copy(ee_hbm.at[pl.ds(row0, 1)],
                              eeb.at[pl.ds(b, 1)], lsem).wait()

    def fire_gather(i):
        b = i % 2
        for k in range(8):
            d = dstb[b, pl.ds(k * 16, 16)]
            s = srcb[b, pl.ds(k * 16, 16)]
            inb = (d >= lo) & (d < lo + HALF)
            ldb[i % 4, pl.ds(k * 16, 16)] = jnp.where(
                inb, d - lo, HALF + (s & 63))
        pltpu.async_copy(feat_hbm.at[srcb.at[b]],
                         rows_v.at[pl.ds(b * 128, 128)], gsem)

    def wait_gather(i):
        b = i % 2
        pltpu.make_async_copy(feat_hbm.at[srcb.at[b]],
                              rows_v.at[pl.ds(b * 128, 128)], gsem).wait()

    load_idx(0)
    wait_idx(0)
    fire_gather(0)

    def win(i, c):
        @pl.when(i + 1 < nwin)
        def _():
            load_idx(i + 1)
        wait_gather(i)

        @pl.when(i + 1 < nwin)
        def _():
            wait_idx(i + 1)
            fire_gather(i + 1)
        b = i % 2

        def scale(g, c2):
            for u in range(4):
                e = b * 128 + g * 4 + u
                a = plsc.load_gather(
                    eeb, [jnp.full((16,), b, jnp.int32),
                          jnp.full((16,), (g * 4 + u), jnp.int32)])
                for cc in range(4):
                    rows_v[e, pl.ds(cc * 16, 16)] = (
                        rows_v[e, pl.ds(cc * 16, 16)] * a)
            return c2

        lax.fori_loop(0, 32, scale, 0)
        pltpu.sync_copy(rows_v.at[pl.ds(b * 128, 128)],
                        out_sh.at[ldb.at[i % 4]], add=True)
        return c

    lax.fori_loop(0, nwin, win, 0)
    plsc.subcore_barrier()
    for c in range(7):
        o = sid * (OT // NS) + c * 224
        pltpu.sync_copy(out_sh.at[pl.ds(o, 224)], rows_v.at[pl.ds(0, 224)])
        pltpu.sync_copy(rows_v.at[pl.ds(0, 224)],
                        zraw_hbm.at[cid, pl.ds(o, 224)])


_k2 = functools.partial(
    pl.kernel,
    _k2_body,
    out_type=jax.ShapeDtypeStruct((NC, OT, OUT), jnp.float32),
    mesh=plsc.VectorSubcoreMesh(core_axis_name="c", subcore_axis_name="s"),
    compiler_params=pltpu.CompilerParams(
        needs_layout_passes=False, use_tc_tiling_on_sc=False),
    scratch_types=[
        pltpu.VMEM((2, 128), jnp.int32),
        pltpu.VMEM((2, 128), jnp.int32),
        pltpu.VMEM((2, 128), jnp.float32),
        pltpu.VMEM((4, 128), jnp.int32),
        pltpu.VMEM((256, OUT), jnp.float32),
        pltpu.VMEM_SHARED((OT, OUT), jnp.float32),
        pltpu.SemaphoreType.DMA,
        pltpu.SemaphoreType.DMA,
    ],
)()


def _k3_body(zr0_ref, zr1_ref, f0_ref, f1_ref, aux_ref, bg_ref,
             w1_ref, b1_ref, w2_ref, o_ref):
    aux = aux_ref[...]
    bgp = bg_ref[...]
    w2 = w2_ref[...]

    def path(zr_ref, f_ref, den, ees, bias):
        x = (zr_ref[0] + ees[:, None] * f_ref[...]) / den[:, None]
        x = x + bias[None, :]
        return jnp.where(x > 0, x, jnp.exp(jnp.minimum(x, 0.0)) - 1.0)

    z0 = path(zr0_ref, f0_ref, aux[:, 0], aux[:, 2], bgp[0])
    z1 = path(zr1_ref, f1_ref, aux[:, 1], aux[:, 3], bgp[1])
    t0 = jnp.tanh(z0 @ w1_ref[...] + b1_ref[...])
    t1 = jnp.tanh(z1 @ w1_ref[...] + b1_ref[...])
    w0 = jnp.sum(t0 * w2[0][None, :], axis=1)
    w1s = jnp.sum(t1 * w2[0][None, :], axis=1)
    m = jnp.maximum(w0, w1s)
    e0 = jnp.exp(w0 - m)
    e1 = jnp.exp(w1s - m)
    b0 = (e0 / (e0 + e1))[:, None]
    b1v = (e1 / (e0 + e1))[:, None]
    o_ref[...] = b0 * z0 + b1v * z1


def _k3(zr0, zr1, f0, f1, aux, bgp, W1, b1r, w2r):
    return pl.pallas_call(
        _k3_body,
        out_shape=jax.ShapeDtypeStruct((N, OUT), jnp.float32),
        grid=(N // BLK,),
        in_specs=[
            pl.BlockSpec((1, BLK, OUT), lambda i: (i // 25, i % 25, 0)),
            pl.BlockSpec((1, BLK, OUT), lambda i: (i // 25, i % 25, 0)),
            pl.BlockSpec((BLK, OUT), lambda i: (i, 0)),
            pl.BlockSpec((BLK, OUT), lambda i: (i, 0)),
            pl.BlockSpec((BLK, 8), lambda i: (i, 0)),
            pl.BlockSpec((8, OUT), lambda i: (0, 0)),
            pl.BlockSpec((OUT, 128), lambda i: (0, 0)),
            pl.BlockSpec((1, 128), lambda i: (0, 0)),
            pl.BlockSpec((1, 128), lambda i: (0, 0)),
        ],
        out_specs=pl.BlockSpec((BLK, OUT), lambda i: (i, 0)),
    )(zr0, zr1, f0, f1, aux, bgp, W1, b1r, w2r)


def kernel(h, edge_index0, edge_index1, Wh, Wg, al, ar, bg, W1, b1, W2):
    pad_i = jnp.arange(PAD, dtype=jnp.int32)
    pad_src = pad_i % 2048
    pad_dst = N + (pad_i % 256)

    def prep(ei):
        srcp = jnp.concatenate([ei[0], pad_src]).reshape(ROWS, 128)
        dstp = jnp.concatenate([ei[1], pad_dst]).reshape(ROWS, 128)
        return srcp, dstp

    src0, dst0 = prep(edge_index0)
    src1, dst1 = prep(edge_index1)

    pvec = jnp.concatenate(
        [al[:, 0, :], ar[:, 0, :], jnp.zeros((4, OUT), jnp.float32)], axis=0)

    feat0, feat1, vecs = _k0(h, Wh, Wg[0], Wg[1], pvec)

    elp0 = jnp.pad(vecs[:, 0], (0, NT - N))
    erp0 = jnp.pad(vecs[:, 1], (0, NT - N))
    elp1 = jnp.pad(vecs[:, 2], (0, NT - N))
    erp1 = jnp.pad(vecs[:, 3], (0, NT - N))

    ee0, sp0 = _k1(elp0, erp0, src0, dst0)
    ee1, sp1 = _k1(elp1, erp1, src1, dst1)

    zraw0 = _k2(feat0, src0, dst0, ee0)
    zraw1 = _k2(feat1, src1, dst1, ee1)

    den0 = sp0[:N] + sp0[NT:NT + N] + vecs[:, 4] + 1e-9
    den1 = sp1[:N] + sp1[NT:NT + N] + vecs[:, 5] + 1e-9
    aux = jnp.stack([den0, den1, vecs[:, 4], vecs[:, 5]], axis=1)
    aux = jnp.concatenate([aux, jnp.zeros((N, 4), jnp.float32)], axis=1)

    bgp = jnp.concatenate([bg, jnp.zeros((6, OUT), jnp.float32)], axis=0)
    return _k3(zraw0, zraw1, feat0, feat1, aux, bgp, W1,
               b1.reshape(1, 128), W2.reshape(1, 128))


# speedup vs baseline: 1.5300x; 1.0004x over previous
import functools

import jax
import jax.numpy as jnp
from jax import lax
from jax.experimental import pallas as pl
from jax.experimental.pallas import tpu as pltpu
from jax.experimental.pallas import tpu_sc as plsc

N = 50000
E = 800000
IN = 128
WS = 64
OUT = 64

NT = 50432
EPAD = 819200
PAD = EPAD - E
ROWS = EPAD // 128

NC = 2
NS = 16
HALF = N // 2
OT = 25088
ZSL = NT // NS

BLK = 1000


def _k0_body(h_ref, wh_ref, wg0_ref, wg1_ref, pv_ref, f0_ref, f1_ref, v_ref):
    wh = h_ref[...] @ wh_ref[...]
    f0 = wh @ wg0_ref[...]
    f1 = wh @ wg1_ref[...]
    f0_ref[...] = f0
    f1_ref[...] = f1
    p = pv_ref[...]
    el0 = jnp.sum(f0 * p[0][None, :], axis=1)
    el1 = jnp.sum(f1 * p[1][None, :], axis=1)
    er0 = jnp.sum(f0 * p[2][None, :], axis=1)
    er1 = jnp.sum(f1 * p[3][None, :], axis=1)
    es0 = el0 + er0
    es1 = el1 + er1
    es0 = jnp.exp(jnp.where(es0 > 0, es0, 0.2 * es0))
    es1 = jnp.exp(jnp.where(es1 > 0, es1, 0.2 * es1))
    z = jnp.zeros_like(el0)
    v_ref[...] = jnp.stack([el0, er0, el1, er1, es0, es1, z, z], axis=1)


def _k0(h, Wh, Wg0, Wg1, pvec):
    return pl.pallas_call(
        _k0_body,
        out_shape=(
            jax.ShapeDtypeStruct((N, OUT), jnp.float32),
            jax.ShapeDtypeStruct((N, OUT), jnp.float32),
            jax.ShapeDtypeStruct((N, 8), jnp.float32),
        ),
        grid=(N // BLK,),
        in_specs=[
            pl.BlockSpec((BLK, IN), lambda i: (i, 0)),
            pl.BlockSpec((IN, WS), lambda i: (0, 0)),
            pl.BlockSpec((WS, OUT), lambda i: (0, 0)),
            pl.BlockSpec((WS, OUT), lambda i: (0, 0)),
            pl.BlockSpec((8, OUT), lambda i: (0, 0)),
        ],
        out_specs=(
            pl.BlockSpec((BLK, OUT), lambda i: (i, 0)),
            pl.BlockSpec((BLK, OUT), lambda i: (i, 0)),
            pl.BlockSpec((BLK, 8), lambda i: (i, 0)),
        ),
    )(h, Wh, Wg0, Wg1, pvec)


def _k1_body(el_hbm, er_hbm, src_hbm, dst_hbm, eexp_hbm, ssum_hbm,
             el_v, er_v, srcb, dstb, eeb, zb, ssum_sh):
    cid = lax.axis_index("c")
    sid = lax.axis_index("s")

    def zloop(i, c):
        zb[pl.ds(i * 16, 16)] = jnp.zeros((16,), jnp.float32)
        return c

    lax.fori_loop(0, ZSL // 16, zloop, 0)
    pltpu.sync_copy(zb, ssum_sh.at[pl.ds(sid * ZSL, ZSL)])
    pltpu.sync_copy(el_hbm, el_v)
    pltpu.sync_copy(er_hbm, er_v)
    plsc.subcore_barrier()

    w = sid * NC + cid
    row0 = w * (ROWS // (NC * NS))

    def win(i, c):
        r = row0 + i * 4
        pltpu.sync_copy(src_hbm.at[pl.ds(r, 4)], srcb)
        pltpu.sync_copy(dst_hbm.at[pl.ds(r, 4)], dstb)
        for j in range(4):
            for k in range(8):
                s = srcb[j, pl.ds(k * 16, 16)]
                d = dstb[j, pl.ds(k * 16, 16)]
                e = plsc.load_gather(el_v, [s]) + plsc.load_gather(er_v, [d])
                e = jnp.where(e > 0, e, 0.2 * e)
                eeb[j, pl.ds(k * 16, 16)] = jnp.exp(e)
        pltpu.sync_copy(eeb, eexp_hbm.at[pl.ds(r, 4)])
        for j in range(4):
            pltpu.sync_copy(eeb.at[j], ssum_sh.at[dstb.at[j]], add=True)
        return c

    lax.fori_loop(0, (ROWS // (NC * NS)) // 4, win, 0)
    plsc.subcore_barrier()
    pltpu.sync_copy(ssum_sh.at[pl.ds(sid * ZSL, ZSL)], zb)
    pltpu.sync_copy(zb, ssum_hbm.at[pl.ds(cid * NT + sid * ZSL, ZSL)])


_k1 = functools.partial(
    pl.kernel,
    _k1_body,
    out_type=(
        jax.ShapeDtypeStruct((ROWS, 128), jnp.float32),
        jax.ShapeDtypeStruct((NC * NT,), jnp.float32),
    ),
    mesh=plsc.VectorSubcoreMesh(core_axis_name="c", subcore_axis_name="s"),
    compiler_params=pltpu.CompilerParams(needs_layout_passes=False),
    scratch_types=[
        pltpu.VMEM((NT,), jnp.float32),
        pltpu.VMEM((NT,), jnp.float32),
        pltpu.VMEM((4, 128), jnp.int32),
        pltpu.VMEM((4, 128), jnp.int32),
        pltpu.VMEM((4, 128), jnp.float32),
        pltpu.VMEM((ZSL,), jnp.float32),
        pltpu.VMEM_SHARED((NT,), jnp.float32),
    ],
)()


def _k2_body(feat_hbm, src_hbm, dst_hbm, ee_hbm, zraw_hbm,
             srcb, dstb, eeb, ldb, rows_v, out_sh,
             lsem, gsem):
    cid = lax.axis_index("c")
    sid = lax.axis_index("s")
    lo = cid * HALF

    def zloop(i, c):
        for cc in range(4):
            rows_v[i, pl.ds(cc * 16, 16)] = jnp.zeros((16,), jnp.float32)
        return c

    lax.fori_loop(0, 256, zloop, 0)
    for c in range(7):
        pltpu.sync_copy(rows_v.at[pl.ds(0, 224)],
                        out_sh.at[pl.ds(sid * (OT // NS) + c * 224, 224)])
    plsc.subcore_barrier()

    row0 = sid * (ROWS // NS)
    nwin = ROWS // NS

    def load_idx(i):
        b = i % 2
        pltpu.async_copy(src_hbm.at[pl.ds(row0 + i, 1)],
                         srcb.at[pl.ds(b, 1)], lsem)
        pltpu.async_copy(dst_hbm.at[pl.ds(row0 + i, 1)],
                         dstb.at[pl.ds(b, 1)], lsem)
        pltpu.async_copy(ee_hbm.at[pl.ds(row0 + i, 1)],
                         eeb.at[pl.ds(b, 1)], lsem)

    def wait_idx(i):
        b = i % 2
        pltpu.make_async_copy(src_hbm.at[pl.ds(row0, 1)],
                              srcb.at[pl.ds(b, 1)], lsem).wait()
        pltpu.make_async_copy(dst_hbm.at[pl.ds(row0, 1)],
                              dstb.at[pl.ds(b, 1)], lsem).wait()
        pltpu.make_async_copy(ee_hbm.at[pl.ds(row0, 1)],
                              eeb.at[pl.ds(b, 1)], lsem).wait()

    def fire_gather(i):
        b = i % 2
        for k in range(8):
            d = dstb[b, pl.ds(k * 16, 16)]
            s = srcb[b, pl.ds(k * 16, 16)]
            inb = (d >= lo) & (d < lo + HALF)
            ldb[i % 4, pl.ds(k * 16, 16)] = jnp.where(
                inb, d - lo, HALF + (s & 63))
        pltpu.async_copy(feat_hbm.at[srcb.at[b]],
                         rows_v.at[pl.ds(b * 128, 128)], gsem)

    def wait_gather(i):
        b = i % 2
        pltpu.make_async_copy(feat_hbm.at[srcb.at[b]],
                              rows_v.at[pl.ds(b * 128, 128)], gsem).wait()

    load_idx(0)
    wait_idx(0)
    fire_gather(0)

    def win(i, c):
        @pl.when(i + 1 < nwin)
        def _():
            load_idx(i + 1)
        wait_gather(i)

        @pl.when(i + 1 < nwin)
        def _():
            wait_idx(i + 1)
            fire_gather(i + 1)
        b = i % 2

        def scale(g, c2):
            for u in range(8):
                e = b * 128 + g * 8 + u
                a = plsc.load_gather(
                    eeb, [jnp.full((16,), b, jnp.int32),
                          jnp.full((16,), (g * 8 + u), jnp.int32)])
                for cc in range(4):
                    rows_v[e, pl.ds(cc * 16, 16)] = (
                        rows_v[e, pl.ds(cc * 16, 16)] * a)
            return c2

        lax.fori_loop(0, 16, scale, 0)
        pltpu.sync_copy(rows_v.at[pl.ds(b * 128, 128)],
                        out_sh.at[ldb.at[i % 4]], add=True)
        return c

    lax.fori_loop(0, nwin, win, 0)
    plsc.subcore_barrier()
    for c in range(7):
        o = sid * (OT // NS) + c * 224
        pltpu.sync_copy(out_sh.at[pl.ds(o, 224)], rows_v.at[pl.ds(0, 224)])
        pltpu.sync_copy(rows_v.at[pl.ds(0, 224)],
                        zraw_hbm.at[cid, pl.ds(o, 224)])


_k2 = functools.partial(
    pl.kernel,
    _k2_body,
    out_type=jax.ShapeDtypeStruct((NC, OT, OUT), jnp.float32),
    mesh=plsc.VectorSubcoreMesh(core_axis_name="c", subcore_axis_name="s"),
    compiler_params=pltpu.CompilerParams(
        needs_layout_passes=False, use_tc_tiling_on_sc=False),
    scratch_types=[
        pltpu.VMEM((2, 128), jnp.int32),
        pltpu.VMEM((2, 128), jnp.int32),
        pltpu.VMEM((2, 128), jnp.float32),
        pltpu.VMEM((4, 128), jnp.int32),
        pltpu.VMEM((256, OUT), jnp.float32),
        pltpu.VMEM_SHARED((OT, OUT), jnp.float32),
        pltpu.SemaphoreType.DMA,
        pltpu.SemaphoreType.DMA,
    ],
)()


def _k3_body(zr0_ref, zr1_ref, f0_ref, f1_ref, aux_ref, bg_ref,
             w1_ref, b1_ref, w2_ref, o_ref):
    aux = aux_ref[...]
    bgp = bg_ref[...]
    w2 = w2_ref[...]

    def path(zr_ref, f_ref, den, ees, bias):
        x = (zr_ref[0] + ees[:, None] * f_ref[...]) / den[:, None]
        x = x + bias[None, :]
        return jnp.where(x > 0, x, jnp.exp(jnp.minimum(x, 0.0)) - 1.0)

    z0 = path(zr0_ref, f0_ref, aux[:, 0], aux[:, 2], bgp[0])
    z1 = path(zr1_ref, f1_ref, aux[:, 1], aux[:, 3], bgp[1])
    t0 = jnp.tanh(z0 @ w1_ref[...] + b1_ref[...])
    t1 = jnp.tanh(z1 @ w1_ref[...] + b1_ref[...])
    w0 = jnp.sum(t0 * w2[0][None, :], axis=1)
    w1s = jnp.sum(t1 * w2[0][None, :], axis=1)
    m = jnp.maximum(w0, w1s)
    e0 = jnp.exp(w0 - m)
    e1 = jnp.exp(w1s - m)
    b0 = (e0 / (e0 + e1))[:, None]
    b1v = (e1 / (e0 + e1))[:, None]
    o_ref[...] = b0 * z0 + b1v * z1


def _k3(zr0, zr1, f0, f1, aux, bgp, W1, b1r, w2r):
    return pl.pallas_call(
        _k3_body,
        out_shape=jax.ShapeDtypeStruct((N, OUT), jnp.float32),
        grid=(N // BLK,),
        in_specs=[
            pl.BlockSpec((1, BLK, OUT), lambda i: (i // 25, i % 25, 0)),
            pl.BlockSpec((1, BLK, OUT), lambda i: (i // 25, i % 25, 0)),
            pl.BlockSpec((BLK, OUT), lambda i: (i, 0)),
            pl.BlockSpec((BLK, OUT), lambda i: (i, 0)),
            pl.BlockSpec((BLK, 8), lambda i: (i, 0)),
            pl.BlockSpec((8, OUT), lambda i: (0, 0)),
            pl.BlockSpec((OUT, 128), lambda i: (0, 0)),
            pl.BlockSpec((1, 128), lambda i: (0, 0)),
            pl.BlockSpec((1, 128), lambda i: (0, 0)),
        ],
        out_specs=pl.BlockSpec((BLK, OUT), lambda i: (i, 0)),
    )(zr0, zr1, f0, f1, aux, bgp, W1, b1r, w2r)


def kernel(h, edge_index0, edge_index1, Wh, Wg, al, ar, bg, W1, b1, W2):
    pad_i = jnp.arange(PAD, dtype=jnp.int32)
    pad_src = pad_i % 2048
    pad_dst = N + (pad_i % 256)

    def prep(ei):
        srcp = jnp.concatenate([ei[0], pad_src]).reshape(ROWS, 128)
        dstp = jnp.concatenate([ei[1], pad_dst]).reshape(ROWS, 128)
        return srcp, dstp

    src0, dst0 = prep(edge_index0)
    src1, dst1 = prep(edge_index1)

    pvec = jnp.concatenate(
        [al[:, 0, :], ar[:, 0, :], jnp.zeros((4, OUT), jnp.float32)], axis=0)

    feat0, feat1, vecs = _k0(h, Wh, Wg[0], Wg[1], pvec)

    elp0 = jnp.pad(vecs[:, 0], (0, NT - N))
    erp0 = jnp.pad(vecs[:, 1], (0, NT - N))
    elp1 = jnp.pad(vecs[:, 2], (0, NT - N))
    erp1 = jnp.pad(vecs[:, 3], (0, NT - N))

    ee0, sp0 = _k1(elp0, erp0, src0, dst0)
    ee1, sp1 = _k1(elp1, erp1, src1, dst1)

    zraw0 = _k2(feat0, src0, dst0, ee0)
    zraw1 = _k2(feat1, src1, dst1, ee1)

    den0 = sp0[:N] + sp0[NT:NT + N] + vecs[:, 4] + 1e-9
    den1 = sp1[:N] + sp1[NT:NT + N] + vecs[:, 5] + 1e-9
    aux = jnp.stack([den0, den1, vecs[:, 4], vecs[:, 5]], axis=1)
    aux = jnp.concatenate([aux, jnp.zeros((N, 4), jnp.float32)], axis=1)

    bgp = jnp.concatenate([bg, jnp.zeros((6, OUT), jnp.float32)], axis=0)
    return _k3(zraw0, zraw1, feat0, feat1, aux, bgp, W1,
               b1.reshape(1, 128), W2.reshape(1, 128))
